# baseline (speedup 1.0000x reference)
import jax
import jax.numpy as jnp
from jax import lax
from jax.experimental import pallas as pl
from jax.experimental.pallas import tpu as pltpu
from jax.experimental.pallas import tpu_sc as plsc

_N = 16384
_C = 1000

_info = plsc.get_sparse_core_info()
_NC = _info.num_cores
_NS = _info.num_subcores
_NW = _NC * _NS
_RPW = _N // _NW
_CH = 32
_NCHUNK = _RPW // _CH
_L = 16


def _body(x_hbm, out_hbm, xv, buf0, buf1, sem0, sem1):
    wid = lax.axis_index("s") * _NC + lax.axis_index("c")
    base = wid * _RPW

    pltpu.sync_copy(x_hbm.at[pl.ds(base, _RPW)], xv)

    bufs = (buf0, buf1)
    sems = (sem0, sem1)
    lane = lax.broadcasted_iota(jnp.int32, (_L,), 0)
    ones = jnp.full((_L,), 1, jnp.int32)
    zeros = jnp.zeros((_L,), jnp.int32)

    def _scatter(buf, k, vals):
        for j in range(_CH // _L):
            cols = xv[pl.ds(k * _CH + j * _L, _L)]
            rows = lane + (j * _L)
            plsc.store_scatter(buf, [rows, cols], vals)

    copies = [None] * _NCHUNK
    for k in range(_NCHUNK):
        b = k % 2
        if k < 2:
            def _zero16(i, _, buf=bufs[b]):
                p = i * (16 * _L) + lane
                for t in range(16):
                    pt = p + t * _L
                    plsc.store_scatter(buf, [pt // _C, pt % _C], zeros)
                return 0
            lax.fori_loop(0, _CH * _C // (16 * _L), _zero16, 0)
        else:
            copies[k - 2].wait()
            _scatter(bufs[b], k - 2, zeros)
        _scatter(bufs[b], k, ones)
        copies[k] = pltpu.async_copy(
            bufs[b], out_hbm.at[pl.ds(base + k * _CH, _CH)], sems[b])
    copies[_NCHUNK - 2].wait()
    copies[_NCHUNK - 1].wait()


def kernel(x):
    mesh = plsc.VectorSubcoreMesh(core_axis_name="c", subcore_axis_name="s")
    f = pl.kernel(
        _body,
        out_type=jax.ShapeDtypeStruct((_N, _C), jnp.int32),
        mesh=mesh,
        scratch_types=[
            pltpu.VMEM((_RPW,), jnp.int32),
            pltpu.VMEM((_CH, _C), jnp.int32),
            pltpu.VMEM((_CH, _C), jnp.int32),
            pltpu.SemaphoreType.DMA,
            pltpu.SemaphoreType.DMA,
        ],
        compiler_params=pltpu.CompilerParams(
            use_tc_tiling_on_sc=True, needs_layout_passes=False),
    )
    return f(x)


# speedup vs baseline: 1.1930x; 1.1930x over previous
import jax
import jax.numpy as jnp
from jax import lax
from jax.experimental import pallas as pl

_N = 16384
_C = 1000
_CP = 1024
_BR = 512


def _onehot_block(x_ref, o_ref):
    col = lax.broadcasted_iota(jnp.int32, (_BR, _CP), 1)
    xv = x_ref[0, 0, :].reshape(_BR, 1)
    o_ref[...] = (xv == col).astype(jnp.int32)


def kernel(x):
    x3 = x.reshape(_N // _BR, 1, _BR)
    out = pl.pallas_call(
        _onehot_block,
        grid=(_N // _BR,),
        in_specs=[pl.BlockSpec((1, 1, _BR), lambda i: (i, 0, 0))],
        out_specs=pl.BlockSpec((_BR, _CP), lambda i: (i, 0)),
        out_shape=jax.ShapeDtypeStruct((_N, _CP), jnp.int32),
    )(x3)
    return out[:, :_C]
